# Initial kernel scaffold; baseline (speedup 1.0000x reference)
import jax
import jax.numpy as jnp
from jax.experimental import pallas as pl


def kernel(x, Wg, bg, W1, b1, W2, b2):
    raise NotImplementedError("write your pallas kernel here")


# speedup vs baseline: 2.7350x; 2.7350x over previous
import functools
import math

import jax
import jax.numpy as jnp
from jax import lax
from jax.experimental import pallas as pl
from jax.experimental.pallas import tpu as pltpu

_LANES = 128


def _gating_body(x_ref, wg_ref, bg_ref, ew_ref):
    x = x_ref[...]
    logits = jnp.dot(x, wg_ref[...], preferred_element_type=jnp.float32)
    logits = logits + bg_ref[...]
    tt = logits.shape[0]
    lane = lax.broadcasted_iota(jnp.int32, (tt, _LANES), 1)
    m1 = jnp.max(logits, axis=-1, keepdims=True)
    e1 = jnp.min(jnp.where(logits == m1, lane, _LANES), axis=-1, keepdims=True)
    masked = jnp.where(lane == e1, -jnp.inf, logits)
    m2 = jnp.max(masked, axis=-1, keepdims=True)
    e2 = jnp.min(jnp.where(masked == m2, lane, _LANES), axis=-1, keepdims=True)
    r = jnp.exp(m2 - m1)
    w1 = 1.0 / (1.0 + r)
    w2 = 1.0 - w1
    out = jnp.where(lane == 0, e1.astype(jnp.float32),
          jnp.where(lane == 1, e2.astype(jnp.float32),
          jnp.where(lane == 2, w1,
          jnp.where(lane == 3, w2, 0.0))))
    ew_ref[...] = out


def _gating(x_flat, Wg, bg):
    T, D = x_flat.shape
    E = Wg.shape[1]
    Wg_p = jnp.pad(Wg, ((0, 0), (0, _LANES - E)))
    bg_p = jnp.pad(bg, (0, _LANES - E), constant_values=-jnp.inf).reshape(1, _LANES)
    ew = pl.pallas_call(
        _gating_body,
        out_shape=jax.ShapeDtypeStruct((T, _LANES), jnp.float32),
    )(x_flat, Wg_p, bg_p)
    e1 = ew[:, 0].astype(jnp.int32)
    e2 = ew[:, 1].astype(jnp.int32)
    w1 = ew[:, 2]
    w2 = ew[:, 3]
    return e1, e2, w1, w2


def _mlp_body(eot_ref, nact_ref, xs_ref, w1_ref, b1_ref, w2_ref, b2_ref,
              ws_ref, ys_ref):
    g = pl.program_id(0)

    @pl.when(g < nact_ref[0])
    def _():
        h = jnp.dot(xs_ref[...], w1_ref[0], preferred_element_type=jnp.float32)
        h = h + b1_ref[0]
        h = 0.5 * h * (1.0 + lax.erf(h * (1.0 / math.sqrt(2.0))))
        y = jnp.dot(h, w2_ref[0], preferred_element_type=jnp.float32)
        y = y + b2_ref[0]
        ys_ref[...] = y * ws_ref[0, 0][:, None]


def _grouped_mlp(xs, W1, b1, W2, b2, w_slot, expert_of_tile, num_active, TS):
    S, D = xs.shape
    E, _, H = W1.shape
    O = W2.shape[2]
    G = S // TS
    grid_spec = pltpu.PrefetchScalarGridSpec(
        num_scalar_prefetch=2,
        grid=(G,),
        in_specs=[
            pl.BlockSpec((TS, D), lambda g, eot, na: (g, 0)),
            pl.BlockSpec((1, D, H), lambda g, eot, na: (eot[g], 0, 0)),
            pl.BlockSpec((1, 1, H), lambda g, eot, na: (eot[g], 0, 0)),
            pl.BlockSpec((1, H, O), lambda g, eot, na: (eot[g], 0, 0)),
            pl.BlockSpec((1, 1, O), lambda g, eot, na: (eot[g], 0, 0)),
            pl.BlockSpec((1, 1, TS), lambda g, eot, na: (g, 0, 0)),
        ],
        out_specs=pl.BlockSpec((TS, O), lambda g, eot, na: (g, 0)),
    )
    return pl.pallas_call(
        _mlp_body,
        grid_spec=grid_spec,
        out_shape=jax.ShapeDtypeStruct((S, O), jnp.float32),
    )(expert_of_tile, num_active, xs, W1.reshape(E, D, H),
      b1.reshape(E, 1, H), W2.reshape(E, H, O), b2.reshape(E, 1, O),
      w_slot.reshape(G, 1, TS))


def kernel(x, Wg, bg, W1, b1, W2, b2):
    B, N, D = x.shape
    T = B * N
    E, _, H = W1.shape
    O = W2.shape[2]
    K = 2
    TS = 256
    P = T * K
    G = P // TS + (E - 1)
    S = G * TS

    x_flat = x.reshape(T, D)
    e1, e2, w1, w2 = _gating(x_flat, Wg, bg)

    e_pair = jnp.stack([e1, e2], axis=1).reshape(P)
    w_pair = jnp.stack([w1, w2], axis=1).reshape(P)
    tok_pair = jax.lax.broadcasted_iota(jnp.int32, (T, K), 0).reshape(P)
    onehot = (e_pair[:, None] == jnp.arange(E, dtype=jnp.int32)[None, :])
    csum = jnp.cumsum(onehot.astype(jnp.int32), axis=0)
    counts = csum[-1]
    rank = jnp.take_along_axis(csum, e_pair[:, None], axis=1)[:, 0] - 1
    tiles_per_e = (counts + TS - 1) // TS
    tile_off = jnp.concatenate(
        [jnp.zeros((1,), jnp.int32), jnp.cumsum(tiles_per_e)[:-1]])
    num_active = jnp.cumsum(tiles_per_e)[-1:].astype(jnp.int32)
    slot = tile_off[e_pair] * TS + rank
    expert_of_tile = jnp.repeat(
        jnp.arange(E, dtype=jnp.int32), tiles_per_e,
        total_repeat_length=G)
    tok_slot = jnp.zeros((S,), jnp.int32).at[slot].set(tok_pair)
    w_slot = jnp.zeros((S,), jnp.float32).at[slot].set(w_pair)

    xs = x_flat[tok_slot]

    ys = _grouped_mlp(xs, W1, b1, W2, b2, w_slot, expert_of_tile,
                      num_active, TS)

    slot2 = slot.reshape(T, K)
    out = ys[slot2[:, 0]] + ys[slot2[:, 1]]
    return out.reshape(B, N, O)
